# baseline (speedup 1.0000x reference)
import functools

import jax
import jax.numpy as jnp
from jax import lax
from jax.experimental import pallas as pl
from jax.experimental.pallas import tpu as pltpu
from jax.experimental.pallas import tpu_sc as plsc

_N_BLOCK = 2048

_NC = 2
_NS = 16
_NW = _NC * _NS
_GATHER_CHUNK = 128


def _argmin_block_kernel(z_ref, cb_ref, zsq_ref, cbsq_ref, idx_ref):
    z = z_ref[...]
    cb = cb_ref[...]
    nb = z.shape[0]
    k = cb.shape[0]

    scores = jax.lax.dot_general(
        z, cb, (((1,), (1,)), ((), ())),
        preferred_element_type=jnp.float32)
    d = (zsq_ref[...] - 2.0 * scores) + cbsq_ref[...]

    rowmin = jnp.min(d, axis=-1, keepdims=True)
    lane = jax.lax.broadcasted_iota(jnp.int32, (nb, k), 1)
    idx = jnp.min(jnp.where(d == rowmin, lane, k), axis=-1)
    idx_ref[...] = idx.reshape(nb // 1024, 1, 1024)


def _tc_argmin(zf, codebook, z_sq, cb_sq):
    n, d_model = zf.shape
    k = codebook.shape[0]
    nb = _N_BLOCK
    return pl.pallas_call(
        _argmin_block_kernel,
        grid=(n // nb,),
        in_specs=[
            pl.BlockSpec((nb, d_model), lambda i: (i, 0)),
            pl.BlockSpec((k, d_model), lambda i: (0, 0)),
            pl.BlockSpec((nb, 1), lambda i: (i, 0)),
            pl.BlockSpec((1, k), lambda i: (0, 0)),
        ],
        out_specs=pl.BlockSpec((nb // 1024, 1, 1024), lambda i: (i, 0, 0)),
        out_shape=jax.ShapeDtypeStruct((n // 1024, 1, 1024), jnp.int32),
        compiler_params=pltpu.CompilerParams(
            dimension_semantics=("arbitrary",)),
    )(zf, codebook, z_sq, cb_sq)


def _sc_gather(codebook, idx_flat, n, d_model):
    bpw = n // _NW
    mesh = plsc.VectorSubcoreMesh(core_axis_name="c", subcore_axis_name="s")

    @functools.partial(
        pl.kernel, mesh=mesh,
        compiler_params=pltpu.CompilerParams(use_tc_tiling_on_sc=False),
        out_type=[
            jax.ShapeDtypeStruct((n, d_model), jnp.float32),
            jax.ShapeDtypeStruct((n, d_model), jnp.float32),
        ],
        scratch_types=[
            pltpu.VMEM((bpw,), jnp.int32),
            pltpu.VMEM((bpw, d_model), jnp.float32),
            pltpu.SemaphoreType.DMA,
        ],
    )
    def sc_kernel(cb_hbm, idx_hbm, out_a, out_b, idx_v, rows_v, sem):
        wid = lax.axis_index("s") * _NC + lax.axis_index("c")
        base = wid * bpw
        pltpu.sync_copy(idx_hbm.at[pl.ds(base, bpw)], idx_v)
        copies = []
        for j in range(0, bpw, _GATHER_CHUNK):
            copies.append(pltpu.async_copy(
                cb_hbm.at[idx_v.at[pl.ds(j, _GATHER_CHUNK)]],
                rows_v.at[pl.ds(j, _GATHER_CHUNK)], sem))
        for c in copies:
            c.wait()
        pltpu.sync_copy(rows_v, out_a.at[pl.ds(base, bpw)])
        pltpu.sync_copy(rows_v, out_b.at[pl.ds(base, bpw)])

    return sc_kernel(codebook, idx_flat)


@jax.jit
def kernel(z, codebook):
    b, t, d_model = z.shape
    n = b * t
    zf = z.reshape(n, d_model)
    z_sq = jnp.sum(zf * zf, axis=-1, keepdims=True)
    cb_sq = jnp.sum(codebook * codebook, axis=-1)[None]

    idx = _tc_argmin(zf, codebook, z_sq, cb_sq)
    zq_st, zq = _sc_gather(codebook, idx.reshape(n), n, d_model)

    return (zq_st.reshape(z.shape), zq.reshape(z.shape),
            idx.reshape(b, t))


# speedup vs baseline: 1.1681x; 1.1681x over previous
import functools

import jax
import jax.numpy as jnp
from jax import lax
from jax.experimental import pallas as pl
from jax.experimental.pallas import tpu as pltpu
from jax.experimental.pallas import tpu_sc as plsc

_N_BLOCK = 1024

_NC = 2
_NS = 16
_NW = _NC * _NS
_GATHER_CHUNK = 128


def _argmin_block_kernel(z_ref, cb_ref, zsq_ref, cbsq_ref, idx_ref):
    z = z_ref[...]
    cb = cb_ref[...]
    nb = z.shape[0]
    k = cb.shape[0]

    scores_t = jax.lax.dot_general(
        cb, z, (((1,), (1,)), ((), ())),
        preferred_element_type=jnp.float32)
    d = (zsq_ref[...] - 2.0 * scores_t) + cbsq_ref[...]

    colmin = jnp.min(d, axis=0, keepdims=True)
    sub = jax.lax.broadcasted_iota(jnp.int32, (k, nb), 0)
    idx = jnp.min(jnp.where(d == colmin, sub, k), axis=0,
                  keepdims=True)
    i = pl.program_id(0)
    idx_ref[pl.ds(i, 1), :] = idx


def _tc_argmin(zf, codebook, z_sq_row, cb_sq_col):
    n, d_model = zf.shape
    k = codebook.shape[0]
    nb = _N_BLOCK
    return pl.pallas_call(
        _argmin_block_kernel,
        grid=(n // nb,),
        in_specs=[
            pl.BlockSpec((nb, d_model), lambda i: (i, 0)),
            pl.BlockSpec((k, d_model), lambda i: (0, 0)),
            pl.BlockSpec((1, nb), lambda i: (0, i)),
            pl.BlockSpec((k, 1), lambda i: (0, 0)),
        ],
        out_specs=pl.BlockSpec((n // nb, nb), lambda i: (0, 0)),
        out_shape=jax.ShapeDtypeStruct((n // nb, nb), jnp.int32),
        compiler_params=pltpu.CompilerParams(
            dimension_semantics=("arbitrary",)),
    )(zf, codebook, z_sq_row, cb_sq_col)


def _sc_gather(codebook, idx_flat, n, d_model):
    bpw = n // _NW
    mesh = plsc.VectorSubcoreMesh(core_axis_name="c", subcore_axis_name="s")

    @functools.partial(
        pl.kernel, mesh=mesh,
        compiler_params=pltpu.CompilerParams(use_tc_tiling_on_sc=False),
        out_type=[
            jax.ShapeDtypeStruct((n, d_model), jnp.float32),
            jax.ShapeDtypeStruct((n, d_model), jnp.float32),
        ],
        scratch_types=[
            pltpu.VMEM((bpw,), jnp.int32),
            pltpu.VMEM((bpw, d_model), jnp.float32),
            pltpu.SemaphoreType.DMA,
        ],
    )
    def sc_kernel(cb_hbm, idx_hbm, out_a, out_b, idx_v, rows_v, sem):
        wid = lax.axis_index("s") * _NC + lax.axis_index("c")
        base = wid * bpw
        pltpu.sync_copy(idx_hbm.at[pl.ds(base, bpw)], idx_v)
        copies = []
        for j in range(0, bpw, _GATHER_CHUNK):
            copies.append(pltpu.async_copy(
                cb_hbm.at[idx_v.at[pl.ds(j, _GATHER_CHUNK)]],
                rows_v.at[pl.ds(j, _GATHER_CHUNK)], sem))
        for c in copies:
            c.wait()
        pltpu.sync_copy(rows_v, out_a.at[pl.ds(base, bpw)])
        pltpu.sync_copy(rows_v, out_b.at[pl.ds(base, bpw)])

    return sc_kernel(codebook, idx_flat)


@jax.jit
def kernel(z, codebook):
    b, t, d_model = z.shape
    n = b * t
    zf = z.reshape(n, d_model)
    z_sq_row = jnp.sum(zf * zf, axis=-1)[None]
    cb_sq_col = jnp.sum(codebook * codebook, axis=-1)[:, None]

    idx = _tc_argmin(zf, codebook, z_sq_row, cb_sq_col)
    zq_st, zq = _sc_gather(codebook, idx.reshape(n), n, d_model)

    return (zq_st.reshape(z.shape), zq.reshape(z.shape),
            idx.reshape(b, t))
